# Initial kernel scaffold; baseline (speedup 1.0000x reference)
import jax
import jax.numpy as jnp
from jax.experimental import pallas as pl


def kernel(x, edge_index, r_scaling_1, r_bias_1, r_scaling_2, r_bias_2, r_scaling_3, r_bias_3, r_scaling_4, r_bias_4, r_scaling_5, r_bias_5):
    raise NotImplementedError("write your pallas kernel here")


# speedup vs baseline: 3.2567x; 3.2567x over previous
import functools

import jax
import jax.numpy as jnp
from jax import lax
from jax.experimental import pallas as pl
from jax.experimental.pallas import tpu as pltpu
from jax.experimental.pallas import tpu_sc as plsc

_N = 10000
_E = 320000
_D = 128
_NEG = _E + _N
_NCAND = _NEG + _NEG // 16 + 1024

_NW = 32
_EPAD = 330240
_RW = _EPAD // _NW
_CH = 240


def _sc_gather_body(ti_hbm, tj_hbm, x_hbm, xi_hbm, xj_hbm,
                    idx_v, rows_v, sem):
    wid = lax.axis_index("s") * 2 + lax.axis_index("c")
    base = wid * _RW

    def step(k, carry):
        off = base + k * _CH
        pltpu.sync_copy(ti_hbm.at[pl.ds(off, _CH)], idx_v)
        pltpu.async_copy(x_hbm.at[idx_v], rows_v, sem).wait()
        pltpu.sync_copy(rows_v, xi_hbm.at[pl.ds(off, _CH)])
        pltpu.sync_copy(tj_hbm.at[pl.ds(off, _CH)], idx_v)
        pltpu.async_copy(x_hbm.at[idx_v], rows_v, sem).wait()
        pltpu.sync_copy(rows_v, xj_hbm.at[pl.ds(off, _CH)])
        return carry

    lax.fori_loop(0, _RW // _CH, step, 0)


_sc_gather = functools.partial(
    pl.kernel,
    out_type=(
        jax.ShapeDtypeStruct((_EPAD, _D), jnp.float32),
        jax.ShapeDtypeStruct((_EPAD, _D), jnp.float32),
    ),
    mesh=plsc.VectorSubcoreMesh(core_axis_name="c", subcore_axis_name="s"),
    scratch_types=[
        pltpu.VMEM((_CH,), jnp.int32),
        pltpu.VMEM((_CH, _D), jnp.float32),
        pltpu.SemaphoreType.DMA,
    ],
)(_sc_gather_body)


_SB = 512


def _score_body(xi_ref, xj_ref, o_ref):
    v = xi_ref[...] * xj_ref[...]
    acc = v[:, 0:8]
    for k in range(1, 16):
        acc = acc + v[:, 8 * k:8 * k + 8]
    t = acc[:, 0:4] + acc[:, 4:8]
    t = t[:, 0:2] + t[:, 2:4]
    o_ref[...] = t[:, 0] + t[:, 1]


def _tc_scores(xi, xj):
    grid = _EPAD // _SB
    return pl.pallas_call(
        _score_body,
        grid=(grid,),
        in_specs=[
            pl.BlockSpec((_SB, _D), lambda i: (i, 0)),
            pl.BlockSpec((_SB, _D), lambda i: (i, 0)),
        ],
        out_specs=pl.BlockSpec((_SB,), lambda i: (i,)),
        out_shape=jax.ShapeDtypeStruct((_EPAD,), jnp.float32),
    )(xi, xj)


def kernel(x, edge_index, r_scaling_1, r_bias_1, r_scaling_2, r_bias_2,
           r_scaling_3, r_bias_3, r_scaling_4, r_bias_4, r_scaling_5,
           r_bias_5):
    row, col = edge_index[0], edge_index[1]
    loop = jnp.arange(_N, dtype=edge_index.dtype)
    keys = jnp.concatenate([row * _N + col, loop * _N + loop])
    pos = jnp.sort(keys)

    key = jax.random.key(42)
    cand = jax.random.randint(key, (_NCAND,), 0, _N * _N)

    idx = jnp.searchsorted(pos, cand)
    idxc = jnp.clip(idx, 0, pos.shape[0] - 1)
    hit = pos[idxc] == cand
    order = jnp.argsort(hit, stable=True)
    cand = cand[order]
    cnt = jnp.maximum(jnp.sum(~hit), 1)
    sel = cand[jnp.arange(_NEG) % cnt]

    tj = (sel // _N).astype(jnp.int32)
    ti = (sel % _N).astype(jnp.int32)
    tj_p = jnp.concatenate([tj, jnp.zeros((_EPAD - _NEG,), jnp.int32)])
    ti_p = jnp.concatenate([ti, jnp.zeros((_EPAD - _NEG,), jnp.int32)])

    xi, xj = _sc_gather(ti_p, tj_p, x)

    s = _tc_scores(xi, xj)[:_NEG]
    s = r_scaling_1 * jax.nn.elu(s) + r_bias_1
    s = r_scaling_2 * jax.nn.elu(s) + r_bias_2
    s = r_scaling_3 * jax.nn.elu(s) + r_bias_3
    s = r_scaling_4 * jax.nn.elu(s) + r_bias_4
    neg_score = r_scaling_5 * jax.nn.elu(s) + r_bias_5

    indices = jnp.argsort(-neg_score)
    sorted_j = jnp.zeros_like(tj).at[indices].set(tj)
    sorted_i = jnp.zeros_like(ti).at[indices].set(ti)
    return jnp.stack([sorted_j, sorted_i])[:, :1000]
